# baseline (speedup 1.0000x reference)
import functools

import jax
import jax.numpy as jnp
from jax import lax
from jax.experimental import pallas as pl
from jax.experimental.pallas import tpu as pltpu
from jax.experimental.pallas import tpu_sc as plsc

BATCH = 32
MAX_SEQ_LEN = 4096
KV_HEAD_DIM = 128
PREFILL_LEN = 2048

BATCH_BLOCK = 8
N_BLOCKS = BATCH // BATCH_BLOCK


def _k_body(x_ref, out_ref):
    out_ref[:, :PREFILL_LEN, :] = x_ref[...]
    out_ref[:, PREFILL_LEN:, :] = jnp.zeros(
        (BATCH_BLOCK, MAX_SEQ_LEN - PREFILL_LEN, KV_HEAD_DIM), jnp.float32
    )


def _tc_k(x):
    in_spec = pl.BlockSpec(
        (BATCH_BLOCK, PREFILL_LEN, KV_HEAD_DIM),
        lambda j: (j, 0, 0),
    )
    out_spec = pl.BlockSpec(
        (BATCH_BLOCK, MAX_SEQ_LEN, KV_HEAD_DIM),
        lambda j: (j, 0, 0),
    )
    return pl.pallas_call(
        _k_body,
        grid=(N_BLOCKS,),
        in_specs=[in_spec],
        out_specs=out_spec,
        out_shape=jax.ShapeDtypeStruct((BATCH, MAX_SEQ_LEN, KV_HEAD_DIM), jnp.float32),
    )(x)


NC = 2
NS = 16
ZROWS = 128
N_ZERO_DMAS = (MAX_SEQ_LEN - PREFILL_LEN) // ZROWS


def _sc_vzero_body(v_out, zb, sem_zs):
    c = lax.axis_index("c")
    s = lax.axis_index("s")
    wid = s * NC + c

    for r in range(ZROWS):
        for col in range(KV_HEAD_DIM // 16):
            zb[r, pl.ds(col * 16, 16)] = jnp.zeros((16,), jnp.float32)

    zeros_out = [
        pltpu.async_copy(
            zb, v_out.at[wid, pl.ds(PREFILL_LEN + i * ZROWS, ZROWS), :], sem_zs
        )
        for i in range(N_ZERO_DMAS)
    ]
    for d in zeros_out:
        d.wait()


def _sc_vzero():
    mesh = plsc.VectorSubcoreMesh(core_axis_name="c", subcore_axis_name="s")
    fn = functools.partial(
        pl.kernel,
        mesh=mesh,
        out_type=jax.ShapeDtypeStruct((BATCH, MAX_SEQ_LEN, KV_HEAD_DIM), jnp.float32),
        scratch_types=[
            pltpu.VMEM((ZROWS, KV_HEAD_DIM), jnp.float32),
            pltpu.SemaphoreType.DMA,
        ],
    )(_sc_vzero_body)
    return fn()


def _vcopy_body(vpart_ref, vx_ref, v_out):
    del vpart_ref
    v_out[...] = vx_ref[...]


def _tc_vcopy(vpart, vx):
    blk = pl.BlockSpec(
        (BATCH_BLOCK, PREFILL_LEN, KV_HEAD_DIM),
        lambda j: (j, 0, 0),
    )
    any_spec = pl.BlockSpec(memory_space=pl.MemorySpace.ANY)
    return pl.pallas_call(
        _vcopy_body,
        grid=(N_BLOCKS,),
        in_specs=[any_spec, blk],
        out_specs=blk,
        out_shape=jax.ShapeDtypeStruct((BATCH, MAX_SEQ_LEN, KV_HEAD_DIM), jnp.float32),
        input_output_aliases={0: 0},
    )(vpart, vx)


def kernel(kx, vx):
    k_cache = _tc_k(kx)
    vpart = _sc_vzero()
    v_cache = _tc_vcopy(vpart, vx)
    return (k_cache, v_cache)
